# baseline (speedup 1.0000x reference)
import functools

import jax
import jax.numpy as jnp
from jax import lax
from jax.experimental import pallas as pl
from jax.experimental.pallas import tpu as pltpu
from jax.experimental.pallas import tpu_sc as plsc

B_, S_, D_ = 1024, 200, 4
EMBED = 64
N = B_ * S_
NW = 32
ROWS_W = N // NW
C = 256
G = C * D_
NCHUNK = ROWS_W // C
SUB = 128
NSUB = G // SUB
LANES = 16
QE = EMBED // LANES

_mesh = plsc.VectorSubcoreMesh(core_axis_name="c", subcore_axis_name="s")


@functools.partial(
    pl.kernel,
    out_type=jax.ShapeDtypeStruct((N, EMBED), jnp.float32),
    mesh=_mesh,
    compiler_params=pltpu.CompilerParams(use_tc_tiling_on_sc=False),
    scratch_types=[
        pltpu.VMEM((NSUB, SUB), jnp.int32),
        pltpu.VMEM((G, EMBED), jnp.float32),
        pltpu.VMEM((C, EMBED), jnp.float32),
        pltpu.SemaphoreType.DMA,
    ],
)
def _emb_pool(idx_hbm, table_hbm, out_hbm, idx_v, gbuf, obuf, sem):
    wid = lax.axis_index("s") * 2 + lax.axis_index("c")
    base = wid * ROWS_W

    def chunk(g, carry):
        rbase = base + g * C
        irow = pl.multiple_of(rbase * D_ // SUB, NSUB)
        pltpu.sync_copy(idx_hbm.at[pl.ds(irow, NSUB)], idx_v)
        descs = [
            pltpu.async_copy(
                table_hbm.at[idx_v.at[j]],
                gbuf.at[pl.ds(j * SUB, SUB)],
                sem,
            )
            for j in range(NSUB)
        ]
        for d in descs:
            d.wait()

        def pool(n, carry2):
            r = n * D_
            for q in range(QE):
                sl = pl.ds(q * LANES, LANES)
                acc = gbuf[r, sl]
                for k in range(1, D_):
                    acc = acc + gbuf[r + k, sl]
                obuf[n, sl] = acc
            return carry2

        lax.fori_loop(0, C, pool, 0, unroll=2)
        pltpu.sync_copy(obuf, out_hbm.at[pl.ds(rbase, C)])
        return carry

    lax.fori_loop(0, NCHUNK, chunk, 0)


def kernel(input, table):
    b, s, d = input.shape
    flat_idx = input.reshape(b * s * d // SUB, SUB)
    out = _emb_pool(flat_idx, table)
    return out.reshape(b, s, EMBED)
